# baseline (speedup 1.0000x reference)
import functools

import jax
import jax.numpy as jnp
from jax import lax
from jax.experimental import pallas as pl
from jax.experimental.pallas import tpu as pltpu
from jax.experimental.pallas import tpu_sc as plsc

N = 10000
E = 320000
D = 128
K = 3
L = 2
STEPS = 2

NC = 2
NS = 16
NW = NC * NS

CHUNK = 50
EPW = E // NW
NCH = EPW // CHUNK
PH = 40
NPH = NCH // PH
NBUF = 4
ECH = 80
NODE_CH = N // ECH
EMB_CPW = -(-NODE_CH // NW)
RPT = 624
RPT_LAST = N - (NS - 1) * RPT

BN = 1000

_SC_MESH = plsc.VectorSubcoreMesh(core_axis_name="c", subcore_axis_name="s")


@functools.partial(
    pl.kernel,
    mesh=_SC_MESH,
    out_type=jax.ShapeDtypeStruct((N, D), jnp.float32),
    scratch_types=[
        pltpu.VMEM((1, ECH), jnp.int32),
        pltpu.VMEM((ECH, D), jnp.float32),
        pltpu.SemaphoreType.DMA,
    ],
)
def _embed_gather_k(emb_hbm, idx_hbm, out_hbm, idx_v, rows_v, sem):
    c = lax.axis_index("c")
    s = lax.axis_index("s")
    w = s * NC + c

    def body(i, carry):
        cid = w * EMB_CPW + i

        @pl.when(cid < NODE_CH)
        def _():
            pltpu.sync_copy(idx_hbm.at[cid], idx_v)
            pltpu.async_copy(emb_hbm.at[idx_v.at[0]], rows_v, sem).wait()
            base = pl.multiple_of(cid * ECH, 8)
            pltpu.sync_copy(rows_v, out_hbm.at[pl.ds(base, ECH)])

        return carry

    lax.fori_loop(0, EMB_CPW, body, 0)


@functools.partial(
    pl.kernel,
    mesh=_SC_MESH,
    out_type=jax.ShapeDtypeStruct((NC, N, D), jnp.float32),
    scratch_types=[
        pltpu.VMEM((PH, CHUNK), jnp.int32),
        pltpu.VMEM((PH, CHUNK), jnp.int32),
        pltpu.VMEM((CHUNK, D), jnp.float32),
        pltpu.VMEM((CHUNK, D), jnp.float32),
        pltpu.VMEM((CHUNK, D), jnp.float32),
        pltpu.VMEM((CHUNK, D), jnp.float32),
        pltpu.SemaphoreType.DMA,
        pltpu.SemaphoreType.DMA,
        pltpu.SemaphoreType.DMA,
        pltpu.SemaphoreType.DMA,
        pltpu.SemaphoreType.DMA,
        pltpu.SemaphoreType.DMA,
        pltpu.SemaphoreType.DMA,
        pltpu.SemaphoreType.DMA,
        pltpu.VMEM_SHARED((N, D), jnp.float32),
    ],
)
def _segsum_k(proj_hbm, src_hbm, dst_hbm, zeros_hbm, out_hbm,
              src_v, dst_v, r0, r1, r2, r3,
              sg0, sg1, sg2, sg3, ss0, ss1, ss2, ss3, acc_sh):
    c = lax.axis_index("c")
    s = lax.axis_index("s")
    w = s * NC + c

    base_r = pl.multiple_of(s * RPT, 8)

    @pl.when(s < NS - 1)
    def _():
        pltpu.sync_copy(zeros_hbm.at[pl.ds(base_r, RPT)],
                        acc_sh.at[pl.ds(base_r, RPT)])

    @pl.when(s == NS - 1)
    def _():
        pltpu.sync_copy(zeros_hbm.at[pl.ds((NS - 1) * RPT, RPT_LAST)],
                        acc_sh.at[pl.ds((NS - 1) * RPT, RPT_LAST)])

    plsc.subcore_barrier()

    rows = (r0, r1, r2, r3)
    sg = (sg0, sg1, sg2, sg3)
    ss = (ss0, ss1, ss2, ss3)

    for p in range(NPH):
        pltpu.sync_copy(src_hbm.at[w, pl.ds(p * PH, PH)], src_v)
        pltpu.sync_copy(dst_hbm.at[w, pl.ds(p * PH, PH)], dst_v)
        for b in range(NBUF):
            pltpu.async_copy(proj_hbm.at[src_v.at[b]], rows[b], sg[b])

        def body(j, carry):
            for b in range(NBUF):
                c = NBUF * j + b
                pltpu.make_async_copy(proj_hbm.at[src_v.at[c]], rows[b],
                                      sg[b]).wait()
                pltpu.async_copy(rows[b], acc_sh.at[dst_v.at[c]], ss[b],
                                 add=True)
            for b in range(NBUF):
                c4 = NBUF * j + b + NBUF

                @pl.when(c4 < PH)
                def _(b=b, c4=c4):
                    pltpu.make_async_copy(rows[b], acc_sh.at[dst_v.at[c4 - NBUF]],
                                          ss[b]).wait()
                    pltpu.async_copy(proj_hbm.at[src_v.at[c4]], rows[b], sg[b])

            return carry

        lax.fori_loop(0, PH // NBUF, body, 0)
        for b in range(NBUF):
            pltpu.make_async_copy(rows[b], acc_sh.at[dst_v.at[PH - NBUF + b]],
                                  ss[b]).wait()

    plsc.subcore_barrier()

    @pl.when(s < NS - 1)
    def _():
        pltpu.sync_copy(acc_sh.at[pl.ds(base_r, RPT)],
                        out_hbm.at[c, pl.ds(base_r, RPT)])

    @pl.when(s == NS - 1)
    def _():
        pltpu.sync_copy(acc_sh.at[pl.ds((NS - 1) * RPT, RPT_LAST)],
                        out_hbm.at[c, pl.ds((NS - 1) * RPT, RPT_LAST)])


def _proj_body(h_ref, wt_ref, b_ref, out_ref):
    out_ref[0] = (
        jnp.dot(h_ref[...], wt_ref[0], preferred_element_type=jnp.float32)
        + b_ref[0]
    )


def _proj(h, wmt, bm3):
    return pl.pallas_call(
        _proj_body,
        grid=(K, N // BN),
        in_specs=[
            pl.BlockSpec((BN, D), lambda k, n: (n, 0)),
            pl.BlockSpec((1, D, D), lambda k, n: (k, 0, 0)),
            pl.BlockSpec((1, 1, D), lambda k, n: (k, 0, 0)),
        ],
        out_specs=pl.BlockSpec((1, BN, D), lambda k, n: (k, n, 0)),
        out_shape=jax.ShapeDtypeStruct((K, N, D), jnp.float32),
    )(h, wmt, bm3)


def _gru_body(acc_ref, h_ref, wih_ref, whh_ref, bih_ref, bhh_ref,
              out_ref, sum_ref):
    a = acc_ref[0] + acc_ref[1]
    h = h_ref[...]
    gi = jnp.dot(a, wih_ref[...], preferred_element_type=jnp.float32) + bih_ref[...]
    gh = jnp.dot(h, whh_ref[...], preferred_element_type=jnp.float32) + bhh_ref[...]
    r = jax.nn.sigmoid(gi[:, :D] + gh[:, :D])
    z = jax.nn.sigmoid(gi[:, D:2 * D] + gh[:, D:2 * D])
    n = jnp.tanh(gi[:, 2 * D:] + r * gh[:, 2 * D:])
    hn = (1.0 - z) * n + z * h
    out_ref[...] = hn
    part = jnp.sum(hn, axis=0, keepdims=True)
    i = pl.program_id(0)

    @pl.when(i == 0)
    def _():
        sum_ref[...] = part

    @pl.when(i != 0)
    def _():
        sum_ref[...] += part

    @pl.when(i == pl.num_programs(0) - 1)
    def _():
        sum_ref[...] *= (1.0 / N)


def _gru(acc2, h, wih_t, whh_t, bih2, bhh2):
    return pl.pallas_call(
        _gru_body,
        grid=(N // BN,),
        in_specs=[
            pl.BlockSpec((NC, BN, D), lambda n: (0, n, 0)),
            pl.BlockSpec((BN, D), lambda n: (n, 0)),
            pl.BlockSpec((D, 3 * D), lambda n: (0, 0)),
            pl.BlockSpec((D, 3 * D), lambda n: (0, 0)),
            pl.BlockSpec((1, 3 * D), lambda n: (0, 0)),
            pl.BlockSpec((1, 3 * D), lambda n: (0, 0)),
        ],
        out_specs=[
            pl.BlockSpec((BN, D), lambda n: (n, 0)),
            pl.BlockSpec((1, D), lambda n: (0, 0)),
        ],
        out_shape=[
            jax.ShapeDtypeStruct((N, D), jnp.float32),
            jax.ShapeDtypeStruct((1, D), jnp.float32),
        ],
    )(acc2, h, wih_t, whh_t, bih2, bhh2)


def _mean_body(h_ref, sum_ref):
    part = jnp.sum(h_ref[...], axis=0, keepdims=True)
    i = pl.program_id(0)

    @pl.when(i == 0)
    def _():
        sum_ref[...] = part

    @pl.when(i != 0)
    def _():
        sum_ref[...] += part

    @pl.when(i == pl.num_programs(0) - 1)
    def _():
        sum_ref[...] *= (1.0 / N)


def _colmean(h):
    return pl.pallas_call(
        _mean_body,
        grid=(N // BN,),
        in_specs=[pl.BlockSpec((BN, D), lambda n: (n, 0))],
        out_specs=pl.BlockSpec((1, D), lambda n: (0, 0)),
        out_shape=jax.ShapeDtypeStruct((1, D), jnp.float32),
    )(h)


def _head_body(agg_ref, w1t_ref, b1_ref, w2_ref, b2_ref, res_ref):
    hidden = jnp.dot(agg_ref[...], w1t_ref[...],
                     preferred_element_type=jnp.float32) + b1_ref[...]
    hidden = jnp.maximum(hidden, 0.0)
    res_ref[...] = jnp.sum(hidden * w2_ref[...], axis=1, keepdims=True) + b2_ref[...]


def _head(agg, w1t, b1r, w2, b2r):
    return pl.pallas_call(
        _head_body,
        in_specs=[
            pl.BlockSpec(agg.shape, lambda: (0, 0)),
            pl.BlockSpec(w1t.shape, lambda: (0, 0)),
            pl.BlockSpec(b1r.shape, lambda: (0, 0)),
            pl.BlockSpec(w2.shape, lambda: (0, 0)),
            pl.BlockSpec(b2r.shape, lambda: (0, 0)),
        ],
        out_specs=pl.BlockSpec((1, 1), lambda: (0, 0)),
        out_shape=jax.ShapeDtypeStruct((1, 1), jnp.float32),
    )(agg, w1t, b1r, w2, b2r)


def kernel(text_idx, edge_src, edge_dst, etypes, emb, Wm, bm,
           W_ih, W_hh, b_ih, b_hh, W1, b1, W2, b2):
    idx2d = text_idx.astype(jnp.int32).reshape(NODE_CH, 1, ECH)
    flat_src = (etypes.astype(jnp.int32) * N + edge_src.astype(jnp.int32))
    src2d = flat_src.reshape(NW, NCH, CHUNK)
    dst2d = edge_dst.astype(jnp.int32).reshape(NW, NCH, CHUNK)
    zeros_nd = jnp.zeros((N, D), jnp.float32)

    h = _embed_gather_k(emb, idx2d)
    means = [_colmean(h)]
    for l in range(L):
        wmt = jnp.transpose(Wm[l], (0, 2, 1))
        bm3 = bm[l].reshape(K, 1, D)
        wih_t = W_ih[l].T
        whh_t = W_hh[l].T
        bih2 = b_ih[l].reshape(1, 3 * D)
        bhh2 = b_hh[l].reshape(1, 3 * D)
        colmean = None
        for _ in range(STEPS):
            proj = _proj(h, wmt, bm3)
            acc2 = _segsum_k(proj.reshape(K * N, D), src2d, dst2d, zeros_nd)
            h, colmean = _gru(acc2, h, wih_t, whh_t, bih2, bhh2)
        means.append(colmean)
    agg = jnp.concatenate(means, axis=1)
    res = _head(agg, W1.T, b1.reshape(1, D), W2, b2.reshape(1, 1))
    return (res, agg)


# speedup vs baseline: 1.1037x; 1.1037x over previous
import functools

import jax
import jax.numpy as jnp
from jax import lax
from jax.experimental import pallas as pl
from jax.experimental.pallas import tpu as pltpu
from jax.experimental.pallas import tpu_sc as plsc

N = 10000
E = 320000
D = 128
K = 3
L = 2
STEPS = 2

NC = 2
NS = 16
NW = NC * NS

CHUNK = 125
EPW = E // NW
NCH = EPW // CHUNK
PH = 40
NPH = NCH // PH
ECH = 80
NODE_CH = N // ECH
EMB_CPW = -(-NODE_CH // NW)
RPT = 624
RPT_LAST = N - (NS - 1) * RPT

BN = 1000

_SC_MESH = plsc.VectorSubcoreMesh(core_axis_name="c", subcore_axis_name="s")


@functools.partial(
    pl.kernel,
    mesh=_SC_MESH,
    out_type=jax.ShapeDtypeStruct((N, D), jnp.float32),
    scratch_types=[
        pltpu.VMEM((1, ECH), jnp.int32),
        pltpu.VMEM((ECH, D), jnp.float32),
        pltpu.SemaphoreType.DMA,
    ],
)
def _embed_gather_k(emb_hbm, idx_hbm, out_hbm, idx_v, rows_v, sem):
    c = lax.axis_index("c")
    s = lax.axis_index("s")
    w = s * NC + c

    def body(i, carry):
        cid = w * EMB_CPW + i

        @pl.when(cid < NODE_CH)
        def _():
            pltpu.sync_copy(idx_hbm.at[cid], idx_v)
            pltpu.async_copy(emb_hbm.at[idx_v.at[0]], rows_v, sem).wait()
            base = pl.multiple_of(cid * ECH, 8)
            pltpu.sync_copy(rows_v, out_hbm.at[pl.ds(base, ECH)])

        return carry

    lax.fori_loop(0, EMB_CPW, body, 0)


@functools.partial(
    pl.kernel,
    mesh=_SC_MESH,
    out_type=jax.ShapeDtypeStruct((NC, N, D), jnp.float32),
    scratch_types=[
        pltpu.VMEM((PH, CHUNK), jnp.int32),
        pltpu.VMEM((PH, CHUNK), jnp.int32),
        pltpu.VMEM((CHUNK, D), jnp.float32),
        pltpu.VMEM((CHUNK, D), jnp.float32),
        pltpu.SemaphoreType.DMA,
        pltpu.SemaphoreType.DMA,
        pltpu.VMEM_SHARED((N, D), jnp.float32),
    ],
)
def _segsum_k(proj_hbm, src_hbm, dst_hbm, zeros_hbm, out_hbm,
              src_v, dst_v, rows_a, rows_b, sem_a, sem_b, acc_sh):
    c = lax.axis_index("c")
    s = lax.axis_index("s")
    w = s * NC + c

    base_r = pl.multiple_of(s * RPT, 8)

    @pl.when(s < NS - 1)
    def _():
        pltpu.sync_copy(zeros_hbm.at[pl.ds(base_r, RPT)],
                        acc_sh.at[pl.ds(base_r, RPT)])

    @pl.when(s == NS - 1)
    def _():
        pltpu.sync_copy(zeros_hbm.at[pl.ds((NS - 1) * RPT, RPT_LAST)],
                        acc_sh.at[pl.ds((NS - 1) * RPT, RPT_LAST)])

    plsc.subcore_barrier()

    for p in range(NPH):
        pltpu.sync_copy(src_hbm.at[w, pl.ds(p * PH, PH)], src_v)
        pltpu.sync_copy(dst_hbm.at[w, pl.ds(p * PH, PH)], dst_v)
        pltpu.async_copy(proj_hbm.at[src_v.at[0]], rows_a, sem_a)
        pltpu.async_copy(proj_hbm.at[src_v.at[1]], rows_b, sem_b)

        def body(i, carry):
            pltpu.make_async_copy(proj_hbm.at[src_v.at[2 * i]], rows_a,
                                  sem_a).wait()
            pltpu.sync_copy(rows_a, acc_sh.at[dst_v.at[2 * i]], add=True)

            @pl.when(2 * i + 2 < PH)
            def _():
                pltpu.async_copy(proj_hbm.at[src_v.at[2 * i + 2]], rows_a,
                                 sem_a)

            pltpu.make_async_copy(proj_hbm.at[src_v.at[2 * i + 1]], rows_b,
                                  sem_b).wait()
            pltpu.sync_copy(rows_b, acc_sh.at[dst_v.at[2 * i + 1]], add=True)

            @pl.when(2 * i + 3 < PH)
            def _():
                pltpu.async_copy(proj_hbm.at[src_v.at[2 * i + 3]], rows_b,
                                 sem_b)

            return carry

        lax.fori_loop(0, PH // 2, body, 0)

    plsc.subcore_barrier()

    @pl.when(s < NS - 1)
    def _():
        pltpu.sync_copy(acc_sh.at[pl.ds(base_r, RPT)],
                        out_hbm.at[c, pl.ds(base_r, RPT)])

    @pl.when(s == NS - 1)
    def _():
        pltpu.sync_copy(acc_sh.at[pl.ds((NS - 1) * RPT, RPT_LAST)],
                        out_hbm.at[c, pl.ds((NS - 1) * RPT, RPT_LAST)])


def _proj_body(h_ref, wt_ref, b_ref, out_ref):
    out_ref[0] = (
        jnp.dot(h_ref[...], wt_ref[0], preferred_element_type=jnp.float32)
        + b_ref[0]
    )


def _proj(h, wmt, bm3):
    return pl.pallas_call(
        _proj_body,
        grid=(K, N // BN),
        in_specs=[
            pl.BlockSpec((BN, D), lambda k, n: (n, 0)),
            pl.BlockSpec((1, D, D), lambda k, n: (k, 0, 0)),
            pl.BlockSpec((1, 1, D), lambda k, n: (k, 0, 0)),
        ],
        out_specs=pl.BlockSpec((1, BN, D), lambda k, n: (k, n, 0)),
        out_shape=jax.ShapeDtypeStruct((K, N, D), jnp.float32),
    )(h, wmt, bm3)


def _gru_body(acc_ref, h_ref, wih_ref, whh_ref, bih_ref, bhh_ref,
              out_ref, sum_ref):
    a = acc_ref[0] + acc_ref[1]
    h = h_ref[...]
    gi = jnp.dot(a, wih_ref[...], preferred_element_type=jnp.float32) + bih_ref[...]
    gh = jnp.dot(h, whh_ref[...], preferred_element_type=jnp.float32) + bhh_ref[...]
    r = jax.nn.sigmoid(gi[:, :D] + gh[:, :D])
    z = jax.nn.sigmoid(gi[:, D:2 * D] + gh[:, D:2 * D])
    n = jnp.tanh(gi[:, 2 * D:] + r * gh[:, 2 * D:])
    hn = (1.0 - z) * n + z * h
    out_ref[...] = hn
    part = jnp.sum(hn, axis=0, keepdims=True)
    i = pl.program_id(0)

    @pl.when(i == 0)
    def _():
        sum_ref[...] = part

    @pl.when(i != 0)
    def _():
        sum_ref[...] += part

    @pl.when(i == pl.num_programs(0) - 1)
    def _():
        sum_ref[...] *= (1.0 / N)


def _gru(acc2, h, wih_t, whh_t, bih2, bhh2):
    return pl.pallas_call(
        _gru_body,
        grid=(N // BN,),
        in_specs=[
            pl.BlockSpec((NC, BN, D), lambda n: (0, n, 0)),
            pl.BlockSpec((BN, D), lambda n: (n, 0)),
            pl.BlockSpec((D, 3 * D), lambda n: (0, 0)),
            pl.BlockSpec((D, 3 * D), lambda n: (0, 0)),
            pl.BlockSpec((1, 3 * D), lambda n: (0, 0)),
            pl.BlockSpec((1, 3 * D), lambda n: (0, 0)),
        ],
        out_specs=[
            pl.BlockSpec((BN, D), lambda n: (n, 0)),
            pl.BlockSpec((1, D), lambda n: (0, 0)),
        ],
        out_shape=[
            jax.ShapeDtypeStruct((N, D), jnp.float32),
            jax.ShapeDtypeStruct((1, D), jnp.float32),
        ],
    )(acc2, h, wih_t, whh_t, bih2, bhh2)


def _mean_body(h_ref, sum_ref):
    part = jnp.sum(h_ref[...], axis=0, keepdims=True)
    i = pl.program_id(0)

    @pl.when(i == 0)
    def _():
        sum_ref[...] = part

    @pl.when(i != 0)
    def _():
        sum_ref[...] += part

    @pl.when(i == pl.num_programs(0) - 1)
    def _():
        sum_ref[...] *= (1.0 / N)


def _colmean(h):
    return pl.pallas_call(
        _mean_body,
        grid=(N // BN,),
        in_specs=[pl.BlockSpec((BN, D), lambda n: (n, 0))],
        out_specs=pl.BlockSpec((1, D), lambda n: (0, 0)),
        out_shape=jax.ShapeDtypeStruct((1, D), jnp.float32),
    )(h)


def _head_body(agg_ref, w1t_ref, b1_ref, w2_ref, b2_ref, res_ref):
    hidden = jnp.dot(agg_ref[...], w1t_ref[...],
                     preferred_element_type=jnp.float32) + b1_ref[...]
    hidden = jnp.maximum(hidden, 0.0)
    res_ref[...] = jnp.sum(hidden * w2_ref[...], axis=1, keepdims=True) + b2_ref[...]


def _head(agg, w1t, b1r, w2, b2r):
    return pl.pallas_call(
        _head_body,
        in_specs=[
            pl.BlockSpec(agg.shape, lambda: (0, 0)),
            pl.BlockSpec(w1t.shape, lambda: (0, 0)),
            pl.BlockSpec(b1r.shape, lambda: (0, 0)),
            pl.BlockSpec(w2.shape, lambda: (0, 0)),
            pl.BlockSpec(b2r.shape, lambda: (0, 0)),
        ],
        out_specs=pl.BlockSpec((1, 1), lambda: (0, 0)),
        out_shape=jax.ShapeDtypeStruct((1, 1), jnp.float32),
    )(agg, w1t, b1r, w2, b2r)


def kernel(text_idx, edge_src, edge_dst, etypes, emb, Wm, bm,
           W_ih, W_hh, b_ih, b_hh, W1, b1, W2, b2):
    idx2d = text_idx.astype(jnp.int32).reshape(NODE_CH, 1, ECH)
    flat_src = (etypes.astype(jnp.int32) * N + edge_src.astype(jnp.int32))
    src2d = flat_src.reshape(NW, NCH, CHUNK)
    dst2d = edge_dst.astype(jnp.int32).reshape(NW, NCH, CHUNK)
    zeros_nd = jnp.zeros((N, D), jnp.float32)

    h = _embed_gather_k(emb, idx2d)
    means = [_colmean(h)]
    for l in range(L):
        wmt = jnp.transpose(Wm[l], (0, 2, 1))
        bm3 = bm[l].reshape(K, 1, D)
        wih_t = W_ih[l].T
        whh_t = W_hh[l].T
        bih2 = b_ih[l].reshape(1, 3 * D)
        bhh2 = b_hh[l].reshape(1, 3 * D)
        colmean = None
        for _ in range(STEPS):
            proj = _proj(h, wmt, bm3)
            acc2 = _segsum_k(proj.reshape(K * N, D), src2d, dst2d, zeros_nd)
            h, colmean = _gru(acc2, h, wih_t, whh_t, bih2, bhh2)
        means.append(colmean)
    agg = jnp.concatenate(means, axis=1)
    res = _head(agg, W1.T, b1.reshape(1, D), W2, b2.reshape(1, 1))
    return (res, agg)


# speedup vs baseline: 1.2127x; 1.0987x over previous
import functools

import jax
import jax.numpy as jnp
from jax import lax
from jax.experimental import pallas as pl
from jax.experimental.pallas import tpu as pltpu
from jax.experimental.pallas import tpu_sc as plsc

N = 10000
E = 320000
D = 128
K = 3
L = 2
STEPS = 2

NC = 2
NS = 16
NW = NC * NS

CHUNK = 125
EPW = E // NW
NCH = EPW // CHUNK
PH = 40
NPH = NCH // PH
ECH = 80
NODE_CH = N // ECH
EMB_CPW = -(-NODE_CH // NW)
RPT = 624
RPT_LAST = N - (NS - 1) * RPT

BN = 1000

_SC_MESH = plsc.VectorSubcoreMesh(core_axis_name="c", subcore_axis_name="s")


@functools.partial(
    pl.kernel,
    mesh=_SC_MESH,
    out_type=jax.ShapeDtypeStruct((N, D), jnp.float32),
    scratch_types=[
        pltpu.VMEM((1, ECH), jnp.int32),
        pltpu.VMEM((ECH, D), jnp.float32),
        pltpu.SemaphoreType.DMA,
    ],
)
def _embed_gather_k(emb_hbm, idx_hbm, out_hbm, idx_v, rows_v, sem):
    c = lax.axis_index("c")
    s = lax.axis_index("s")
    w = s * NC + c

    def body(i, carry):
        cid = w * EMB_CPW + i

        @pl.when(cid < NODE_CH)
        def _():
            pltpu.sync_copy(idx_hbm.at[cid], idx_v)
            pltpu.async_copy(emb_hbm.at[idx_v.at[0]], rows_v, sem).wait()
            base = pl.multiple_of(cid * ECH, 8)
            pltpu.sync_copy(rows_v, out_hbm.at[pl.ds(base, ECH)])

        return carry

    lax.fori_loop(0, EMB_CPW, body, 0)


@functools.partial(
    pl.kernel,
    mesh=_SC_MESH,
    out_type=jax.ShapeDtypeStruct((NC, N, D), jnp.float32),
    scratch_types=[
        pltpu.VMEM((PH, CHUNK), jnp.int32),
        pltpu.VMEM((PH, CHUNK), jnp.int32),
        pltpu.VMEM((CHUNK, D), jnp.float32),
        pltpu.VMEM((CHUNK, D), jnp.float32),
        pltpu.SemaphoreType.DMA,
        pltpu.SemaphoreType.DMA,
        pltpu.VMEM_SHARED((N, D), jnp.float32),
    ],
)
def _segsum_k(proj_hbm, src_hbm, dst_hbm, zeros_hbm, out_hbm,
              src_v, dst_v, rows_a, rows_b, sem_a, sem_b, acc_sh):
    c = lax.axis_index("c")
    s = lax.axis_index("s")
    w = s * NC + c

    base_r = pl.multiple_of(s * RPT, 8)

    @pl.when(s < NS - 1)
    def _():
        pltpu.sync_copy(zeros_hbm.at[pl.ds(base_r, RPT)],
                        acc_sh.at[pl.ds(base_r, RPT)])

    @pl.when(s == NS - 1)
    def _():
        pltpu.sync_copy(zeros_hbm.at[pl.ds((NS - 1) * RPT, RPT_LAST)],
                        acc_sh.at[pl.ds((NS - 1) * RPT, RPT_LAST)])

    plsc.subcore_barrier()

    for p in range(NPH):
        pltpu.sync_copy(src_hbm.at[w, pl.ds(p * PH, PH)], src_v)
        pltpu.sync_copy(dst_hbm.at[w, pl.ds(p * PH, PH)], dst_v)
        pltpu.async_copy(proj_hbm.at[src_v.at[0]], rows_a, sem_a)
        pltpu.async_copy(proj_hbm.at[src_v.at[1]], rows_b, sem_b)

        def body(i, carry):
            pltpu.make_async_copy(proj_hbm.at[src_v.at[2 * i]], rows_a,
                                  sem_a).wait()
            pltpu.sync_copy(rows_a, acc_sh.at[dst_v.at[2 * i]], add=True)

            @pl.when(2 * i + 2 < PH)
            def _():
                pltpu.async_copy(proj_hbm.at[src_v.at[2 * i + 2]], rows_a,
                                 sem_a)

            pltpu.make_async_copy(proj_hbm.at[src_v.at[2 * i + 1]], rows_b,
                                  sem_b).wait()
            pltpu.sync_copy(rows_b, acc_sh.at[dst_v.at[2 * i + 1]], add=True)

            @pl.when(2 * i + 3 < PH)
            def _():
                pltpu.async_copy(proj_hbm.at[src_v.at[2 * i + 3]], rows_b,
                                 sem_b)

            return carry

        lax.fori_loop(0, PH // 2, body, 0)

    plsc.subcore_barrier()

    @pl.when(s < NS - 1)
    def _():
        pltpu.sync_copy(acc_sh.at[pl.ds(base_r, RPT)],
                        out_hbm.at[c, pl.ds(base_r, RPT)])

    @pl.when(s == NS - 1)
    def _():
        pltpu.sync_copy(acc_sh.at[pl.ds((NS - 1) * RPT, RPT_LAST)],
                        out_hbm.at[c, pl.ds((NS - 1) * RPT, RPT_LAST)])


def _proj_first_body(h_ref, wt_ref, b_ref, out_ref, sum_ref):
    k = pl.program_id(0)
    n = pl.program_id(1)
    out_ref[0] = (
        jnp.dot(h_ref[...], wt_ref[0], preferred_element_type=jnp.float32)
        + b_ref[0]
    )

    @pl.when(jnp.logical_and(k == 0, n == 0))
    def _():
        sum_ref[...] = jnp.sum(h_ref[...], axis=0, keepdims=True)

    @pl.when(jnp.logical_and(k == 0, n != 0))
    def _():
        sum_ref[...] += jnp.sum(h_ref[...], axis=0, keepdims=True)

    @pl.when(jnp.logical_and(k == 0, n == pl.num_programs(1) - 1))
    def _():
        sum_ref[...] *= (1.0 / N)


def _proj_first(h, wmt, bm3):
    return pl.pallas_call(
        _proj_first_body,
        grid=(K, N // BN),
        in_specs=[
            pl.BlockSpec((BN, D), lambda k, n: (n, 0)),
            pl.BlockSpec((1, D, D), lambda k, n: (k, 0, 0)),
            pl.BlockSpec((1, 1, D), lambda k, n: (k, 0, 0)),
        ],
        out_specs=[
            pl.BlockSpec((1, BN, D), lambda k, n: (k, n, 0)),
            pl.BlockSpec((1, D), lambda k, n: (0, 0)),
        ],
        out_shape=[
            jax.ShapeDtypeStruct((K, N, D), jnp.float32),
            jax.ShapeDtypeStruct((1, D), jnp.float32),
        ],
    )(h, wmt, bm3)


def _gru_math(acc_ref, h_ref, wih_ref, whh_ref, bih_ref, bhh_ref):
    a = acc_ref[0] + acc_ref[1]
    h = h_ref[...]
    gi = jnp.dot(a, wih_ref[...], preferred_element_type=jnp.float32) + bih_ref[...]
    gh = jnp.dot(h, whh_ref[...], preferred_element_type=jnp.float32) + bhh_ref[...]
    r = jax.nn.sigmoid(gi[:, :D] + gh[:, :D])
    z = jax.nn.sigmoid(gi[:, D:2 * D] + gh[:, D:2 * D])
    n = jnp.tanh(gi[:, 2 * D:] + r * gh[:, 2 * D:])
    return (1.0 - z) * n + z * h


def _colsum_update(sum_ref, hn, i):
    part = jnp.sum(hn, axis=0, keepdims=True)

    @pl.when(i == 0)
    def _():
        sum_ref[...] = part

    @pl.when(i != 0)
    def _():
        sum_ref[...] += part

    @pl.when(i == pl.num_programs(0) - 1)
    def _():
        sum_ref[...] *= (1.0 / N)


def _gru_body(acc_ref, h_ref, wih_ref, whh_ref, bih_ref, bhh_ref,
              out_ref, sum_ref):
    hn = _gru_math(acc_ref, h_ref, wih_ref, whh_ref, bih_ref, bhh_ref)
    out_ref[...] = hn
    _colsum_update(sum_ref, hn, pl.program_id(0))


def _gru_proj_body(acc_ref, h_ref, wih_ref, whh_ref, bih_ref, bhh_ref,
                   wt_ref, bm_ref, out_ref, sum_ref, proj_ref):
    hn = _gru_math(acc_ref, h_ref, wih_ref, whh_ref, bih_ref, bhh_ref)
    out_ref[...] = hn
    _colsum_update(sum_ref, hn, pl.program_id(0))
    for k in range(K):
        proj_ref[k] = (
            jnp.dot(hn, wt_ref[k], preferred_element_type=jnp.float32)
            + bm_ref[k]
        )


def _gru_proj(acc2, h, wih_t, whh_t, bih2, bhh2, wmt, bm3):
    return pl.pallas_call(
        _gru_proj_body,
        grid=(N // BN,),
        in_specs=[
            pl.BlockSpec((NC, BN, D), lambda n: (0, n, 0)),
            pl.BlockSpec((BN, D), lambda n: (n, 0)),
            pl.BlockSpec((D, 3 * D), lambda n: (0, 0)),
            pl.BlockSpec((D, 3 * D), lambda n: (0, 0)),
            pl.BlockSpec((1, 3 * D), lambda n: (0, 0)),
            pl.BlockSpec((1, 3 * D), lambda n: (0, 0)),
            pl.BlockSpec((K, D, D), lambda n: (0, 0, 0)),
            pl.BlockSpec((K, 1, D), lambda n: (0, 0, 0)),
        ],
        out_specs=[
            pl.BlockSpec((BN, D), lambda n: (n, 0)),
            pl.BlockSpec((1, D), lambda n: (0, 0)),
            pl.BlockSpec((K, BN, D), lambda n: (0, n, 0)),
        ],
        out_shape=[
            jax.ShapeDtypeStruct((N, D), jnp.float32),
            jax.ShapeDtypeStruct((1, D), jnp.float32),
            jax.ShapeDtypeStruct((K, N, D), jnp.float32),
        ],
    )(acc2, h, wih_t, whh_t, bih2, bhh2, wmt, bm3)


def _gru(acc2, h, wih_t, whh_t, bih2, bhh2):
    return pl.pallas_call(
        _gru_body,
        grid=(N // BN,),
        in_specs=[
            pl.BlockSpec((NC, BN, D), lambda n: (0, n, 0)),
            pl.BlockSpec((BN, D), lambda n: (n, 0)),
            pl.BlockSpec((D, 3 * D), lambda n: (0, 0)),
            pl.BlockSpec((D, 3 * D), lambda n: (0, 0)),
            pl.BlockSpec((1, 3 * D), lambda n: (0, 0)),
            pl.BlockSpec((1, 3 * D), lambda n: (0, 0)),
        ],
        out_specs=[
            pl.BlockSpec((BN, D), lambda n: (n, 0)),
            pl.BlockSpec((1, D), lambda n: (0, 0)),
        ],
        out_shape=[
            jax.ShapeDtypeStruct((N, D), jnp.float32),
            jax.ShapeDtypeStruct((1, D), jnp.float32),
        ],
    )(acc2, h, wih_t, whh_t, bih2, bhh2)


def _head_body(agg_ref, w1t_ref, b1_ref, w2_ref, b2_ref, res_ref):
    hidden = jnp.dot(agg_ref[...], w1t_ref[...],
                     preferred_element_type=jnp.float32) + b1_ref[...]
    hidden = jnp.maximum(hidden, 0.0)
    res_ref[...] = jnp.sum(hidden * w2_ref[...], axis=1, keepdims=True) + b2_ref[...]


def _head(agg, w1t, b1r, w2, b2r):
    return pl.pallas_call(
        _head_body,
        in_specs=[
            pl.BlockSpec(agg.shape, lambda: (0, 0)),
            pl.BlockSpec(w1t.shape, lambda: (0, 0)),
            pl.BlockSpec(b1r.shape, lambda: (0, 0)),
            pl.BlockSpec(w2.shape, lambda: (0, 0)),
            pl.BlockSpec(b2r.shape, lambda: (0, 0)),
        ],
        out_specs=pl.BlockSpec((1, 1), lambda: (0, 0)),
        out_shape=jax.ShapeDtypeStruct((1, 1), jnp.float32),
    )(agg, w1t, b1r, w2, b2r)


def kernel(text_idx, edge_src, edge_dst, etypes, emb, Wm, bm,
           W_ih, W_hh, b_ih, b_hh, W1, b1, W2, b2):
    idx2d = text_idx.astype(jnp.int32).reshape(NODE_CH, 1, ECH)
    flat_src = (etypes.astype(jnp.int32) * N + edge_src.astype(jnp.int32))
    src2d = flat_src.reshape(NW, NCH, CHUNK)
    dst2d = edge_dst.astype(jnp.int32).reshape(NW, NCH, CHUNK)
    zeros_nd = jnp.zeros((N, D), jnp.float32)

    wmt = [jnp.transpose(Wm[l], (0, 2, 1)) for l in range(L)]
    bm3 = [bm[l].reshape(K, 1, D) for l in range(L)]
    wih_t = [W_ih[l].T for l in range(L)]
    whh_t = [W_hh[l].T for l in range(L)]
    bih2 = [b_ih[l].reshape(1, 3 * D) for l in range(L)]
    bhh2 = [b_hh[l].reshape(1, 3 * D) for l in range(L)]

    h = _embed_gather_k(emb, idx2d)
    proj, m0 = _proj_first(h, wmt[0], bm3[0])
    means = [m0]
    rounds = L * STEPS
    for r in range(rounds):
        l = r // 2
        acc2 = _segsum_k(proj.reshape(K * N, D), src2d, dst2d, zeros_nd)
        if r < rounds - 1:
            ln = (r + 1) // 2
            h, colmean, proj = _gru_proj(acc2, h, wih_t[l], whh_t[l],
                                         bih2[l], bhh2[l], wmt[ln], bm3[ln])
        else:
            h, colmean = _gru(acc2, h, wih_t[l], whh_t[l], bih2[l], bhh2[l])
        if r % STEPS == STEPS - 1:
            means.append(colmean)
    agg = jnp.concatenate(means, axis=1)
    res = _head(agg, W1.T, b1.reshape(1, D), W2, b2.reshape(1, 1))
    return (res, agg)


# speedup vs baseline: 1.2605x; 1.0395x over previous
import functools

import jax
import jax.numpy as jnp
from jax import lax
from jax.experimental import pallas as pl
from jax.experimental.pallas import tpu as pltpu
from jax.experimental.pallas import tpu_sc as plsc

N = 10000
E = 320000
D = 128
K = 3
L = 2
STEPS = 2

NC = 2
NS = 16
NW = NC * NS

CHUNK = 125
EPW = E // NW
NCH = EPW // CHUNK
PH = 40
NPH = NCH // PH
ECH = 80
NODE_CH = N // ECH
EMB_CPW = -(-NODE_CH // NW)
RPT = 624
RPT_LAST = N - (NS - 1) * RPT

BN = 2000

_SC_MESH = plsc.VectorSubcoreMesh(core_axis_name="c", subcore_axis_name="s")


@functools.partial(
    pl.kernel,
    mesh=_SC_MESH,
    out_type=jax.ShapeDtypeStruct((N, D), jnp.float32),
    scratch_types=[
        pltpu.VMEM((1, ECH), jnp.int32),
        pltpu.VMEM((ECH, D), jnp.float32),
        pltpu.SemaphoreType.DMA,
    ],
)
def _embed_gather_k(emb_hbm, idx_hbm, out_hbm, idx_v, rows_v, sem):
    c = lax.axis_index("c")
    s = lax.axis_index("s")
    w = s * NC + c

    def body(i, carry):
        cid = w * EMB_CPW + i

        @pl.when(cid < NODE_CH)
        def _():
            pltpu.sync_copy(idx_hbm.at[cid], idx_v)
            pltpu.async_copy(emb_hbm.at[idx_v.at[0]], rows_v, sem).wait()
            base = pl.multiple_of(cid * ECH, 8)
            pltpu.sync_copy(rows_v, out_hbm.at[pl.ds(base, ECH)])

        return carry

    lax.fori_loop(0, EMB_CPW, body, 0)


@functools.partial(
    pl.kernel,
    mesh=_SC_MESH,
    out_type=jax.ShapeDtypeStruct((NC, N, D), jnp.float32),
    scratch_types=[
        pltpu.VMEM((PH, CHUNK), jnp.int32),
        pltpu.VMEM((PH, CHUNK), jnp.int32),
        pltpu.VMEM((CHUNK, D), jnp.float32),
        pltpu.VMEM((CHUNK, D), jnp.float32),
        pltpu.SemaphoreType.DMA,
        pltpu.SemaphoreType.DMA,
        pltpu.VMEM_SHARED((N, D), jnp.float32),
    ],
)
def _segsum_k(proj_hbm, src_hbm, dst_hbm, zeros_hbm, out_hbm,
              src_v, dst_v, rows_a, rows_b, sem_a, sem_b, acc_sh):
    c = lax.axis_index("c")
    s = lax.axis_index("s")
    w = s * NC + c

    base_r = pl.multiple_of(s * RPT, 8)

    @pl.when(s < NS - 1)
    def _():
        pltpu.sync_copy(zeros_hbm.at[pl.ds(base_r, RPT)],
                        acc_sh.at[pl.ds(base_r, RPT)])

    @pl.when(s == NS - 1)
    def _():
        pltpu.sync_copy(zeros_hbm.at[pl.ds((NS - 1) * RPT, RPT_LAST)],
                        acc_sh.at[pl.ds((NS - 1) * RPT, RPT_LAST)])

    plsc.subcore_barrier()

    for p in range(NPH):
        pltpu.sync_copy(src_hbm.at[w, pl.ds(p * PH, PH)], src_v)
        pltpu.sync_copy(dst_hbm.at[w, pl.ds(p * PH, PH)], dst_v)
        pltpu.async_copy(proj_hbm.at[src_v.at[0]], rows_a, sem_a)
        pltpu.async_copy(proj_hbm.at[src_v.at[1]], rows_b, sem_b)

        def body(i, carry):
            pltpu.make_async_copy(proj_hbm.at[src_v.at[2 * i]], rows_a,
                                  sem_a).wait()
            pltpu.sync_copy(rows_a, acc_sh.at[dst_v.at[2 * i]], add=True)

            @pl.when(2 * i + 2 < PH)
            def _():
                pltpu.async_copy(proj_hbm.at[src_v.at[2 * i + 2]], rows_a,
                                 sem_a)

            pltpu.make_async_copy(proj_hbm.at[src_v.at[2 * i + 1]], rows_b,
                                  sem_b).wait()
            pltpu.sync_copy(rows_b, acc_sh.at[dst_v.at[2 * i + 1]], add=True)

            @pl.when(2 * i + 3 < PH)
            def _():
                pltpu.async_copy(proj_hbm.at[src_v.at[2 * i + 3]], rows_b,
                                 sem_b)

            return carry

        lax.fori_loop(0, PH // 2, body, 0)

    plsc.subcore_barrier()

    @pl.when(s < NS - 1)
    def _():
        pltpu.sync_copy(acc_sh.at[pl.ds(base_r, RPT)],
                        out_hbm.at[c, pl.ds(base_r, RPT)])

    @pl.when(s == NS - 1)
    def _():
        pltpu.sync_copy(acc_sh.at[pl.ds((NS - 1) * RPT, RPT_LAST)],
                        out_hbm.at[c, pl.ds((NS - 1) * RPT, RPT_LAST)])


def _proj_first_body(h_ref, wt_ref, b_ref, out_ref, sum_ref):
    k = pl.program_id(0)
    n = pl.program_id(1)
    out_ref[0] = (
        jnp.dot(h_ref[...], wt_ref[0], preferred_element_type=jnp.float32)
        + b_ref[0]
    )

    @pl.when(jnp.logical_and(k == 0, n == 0))
    def _():
        sum_ref[...] = jnp.sum(h_ref[...], axis=0, keepdims=True)

    @pl.when(jnp.logical_and(k == 0, n != 0))
    def _():
        sum_ref[...] += jnp.sum(h_ref[...], axis=0, keepdims=True)

    @pl.when(jnp.logical_and(k == 0, n == pl.num_programs(1) - 1))
    def _():
        sum_ref[...] *= (1.0 / N)


def _proj_first(h, wmt, bm3):
    return pl.pallas_call(
        _proj_first_body,
        grid=(K, N // BN),
        in_specs=[
            pl.BlockSpec((BN, D), lambda k, n: (n, 0)),
            pl.BlockSpec((1, D, D), lambda k, n: (k, 0, 0)),
            pl.BlockSpec((1, 1, D), lambda k, n: (k, 0, 0)),
        ],
        out_specs=[
            pl.BlockSpec((1, BN, D), lambda k, n: (k, n, 0)),
            pl.BlockSpec((1, D), lambda k, n: (0, 0)),
        ],
        out_shape=[
            jax.ShapeDtypeStruct((K, N, D), jnp.float32),
            jax.ShapeDtypeStruct((1, D), jnp.float32),
        ],
    )(h, wmt, bm3)


def _gru_math(acc_ref, h_ref, wih_ref, whh_ref, bih_ref, bhh_ref):
    a = acc_ref[0] + acc_ref[1]
    h = h_ref[...]
    gi = jnp.dot(a, wih_ref[...], preferred_element_type=jnp.float32) + bih_ref[...]
    gh = jnp.dot(h, whh_ref[...], preferred_element_type=jnp.float32) + bhh_ref[...]
    r = jax.nn.sigmoid(gi[:, :D] + gh[:, :D])
    z = jax.nn.sigmoid(gi[:, D:2 * D] + gh[:, D:2 * D])
    n = jnp.tanh(gi[:, 2 * D:] + r * gh[:, 2 * D:])
    return (1.0 - z) * n + z * h


def _colsum_update(sum_ref, hn, i):
    part = jnp.sum(hn, axis=0, keepdims=True)

    @pl.when(i == 0)
    def _():
        sum_ref[...] = part

    @pl.when(i != 0)
    def _():
        sum_ref[...] += part

    @pl.when(i == pl.num_programs(0) - 1)
    def _():
        sum_ref[...] *= (1.0 / N)


def _gru_body(acc_ref, h_ref, wih_ref, whh_ref, bih_ref, bhh_ref,
              out_ref, sum_ref):
    hn = _gru_math(acc_ref, h_ref, wih_ref, whh_ref, bih_ref, bhh_ref)
    out_ref[...] = hn
    _colsum_update(sum_ref, hn, pl.program_id(0))


def _gru_proj_body(acc_ref, h_ref, wih_ref, whh_ref, bih_ref, bhh_ref,
                   wt_ref, bm_ref, out_ref, sum_ref, proj_ref):
    hn = _gru_math(acc_ref, h_ref, wih_ref, whh_ref, bih_ref, bhh_ref)
    out_ref[...] = hn
    _colsum_update(sum_ref, hn, pl.program_id(0))
    for k in range(K):
        proj_ref[k] = (
            jnp.dot(hn, wt_ref[k], preferred_element_type=jnp.float32)
            + bm_ref[k]
        )


def _gru_proj(acc2, h, wih_t, whh_t, bih2, bhh2, wmt, bm3):
    return pl.pallas_call(
        _gru_proj_body,
        grid=(N // BN,),
        in_specs=[
            pl.BlockSpec((NC, BN, D), lambda n: (0, n, 0)),
            pl.BlockSpec((BN, D), lambda n: (n, 0)),
            pl.BlockSpec((D, 3 * D), lambda n: (0, 0)),
            pl.BlockSpec((D, 3 * D), lambda n: (0, 0)),
            pl.BlockSpec((1, 3 * D), lambda n: (0, 0)),
            pl.BlockSpec((1, 3 * D), lambda n: (0, 0)),
            pl.BlockSpec((K, D, D), lambda n: (0, 0, 0)),
            pl.BlockSpec((K, 1, D), lambda n: (0, 0, 0)),
        ],
        out_specs=[
            pl.BlockSpec((BN, D), lambda n: (n, 0)),
            pl.BlockSpec((1, D), lambda n: (0, 0)),
            pl.BlockSpec((K, BN, D), lambda n: (0, n, 0)),
        ],
        out_shape=[
            jax.ShapeDtypeStruct((N, D), jnp.float32),
            jax.ShapeDtypeStruct((1, D), jnp.float32),
            jax.ShapeDtypeStruct((K, N, D), jnp.float32),
        ],
    )(acc2, h, wih_t, whh_t, bih2, bhh2, wmt, bm3)


def _gru(acc2, h, wih_t, whh_t, bih2, bhh2):
    return pl.pallas_call(
        _gru_body,
        grid=(N // BN,),
        in_specs=[
            pl.BlockSpec((NC, BN, D), lambda n: (0, n, 0)),
            pl.BlockSpec((BN, D), lambda n: (n, 0)),
            pl.BlockSpec((D, 3 * D), lambda n: (0, 0)),
            pl.BlockSpec((D, 3 * D), lambda n: (0, 0)),
            pl.BlockSpec((1, 3 * D), lambda n: (0, 0)),
            pl.BlockSpec((1, 3 * D), lambda n: (0, 0)),
        ],
        out_specs=[
            pl.BlockSpec((BN, D), lambda n: (n, 0)),
            pl.BlockSpec((1, D), lambda n: (0, 0)),
        ],
        out_shape=[
            jax.ShapeDtypeStruct((N, D), jnp.float32),
            jax.ShapeDtypeStruct((1, D), jnp.float32),
        ],
    )(acc2, h, wih_t, whh_t, bih2, bhh2)


def _head_body(agg_ref, w1t_ref, b1_ref, w2_ref, b2_ref, res_ref):
    hidden = jnp.dot(agg_ref[...], w1t_ref[...],
                     preferred_element_type=jnp.float32) + b1_ref[...]
    hidden = jnp.maximum(hidden, 0.0)
    res_ref[...] = jnp.sum(hidden * w2_ref[...], axis=1, keepdims=True) + b2_ref[...]


def _head(agg, w1t, b1r, w2, b2r):
    return pl.pallas_call(
        _head_body,
        in_specs=[
            pl.BlockSpec(agg.shape, lambda: (0, 0)),
            pl.BlockSpec(w1t.shape, lambda: (0, 0)),
            pl.BlockSpec(b1r.shape, lambda: (0, 0)),
            pl.BlockSpec(w2.shape, lambda: (0, 0)),
            pl.BlockSpec(b2r.shape, lambda: (0, 0)),
        ],
        out_specs=pl.BlockSpec((1, 1), lambda: (0, 0)),
        out_shape=jax.ShapeDtypeStruct((1, 1), jnp.float32),
    )(agg, w1t, b1r, w2, b2r)


def kernel(text_idx, edge_src, edge_dst, etypes, emb, Wm, bm,
           W_ih, W_hh, b_ih, b_hh, W1, b1, W2, b2):
    idx2d = text_idx.astype(jnp.int32).reshape(NODE_CH, 1, ECH)
    flat_src = (etypes.astype(jnp.int32) * N + edge_src.astype(jnp.int32))
    src2d = flat_src.reshape(NW, NCH, CHUNK)
    dst2d = edge_dst.astype(jnp.int32).reshape(NW, NCH, CHUNK)
    zeros_nd = jnp.zeros((N, D), jnp.float32)

    wmt = [jnp.transpose(Wm[l], (0, 2, 1)) for l in range(L)]
    bm3 = [bm[l].reshape(K, 1, D) for l in range(L)]
    wih_t = [W_ih[l].T for l in range(L)]
    whh_t = [W_hh[l].T for l in range(L)]
    bih2 = [b_ih[l].reshape(1, 3 * D) for l in range(L)]
    bhh2 = [b_hh[l].reshape(1, 3 * D) for l in range(L)]

    h = _embed_gather_k(emb, idx2d)
    proj, m0 = _proj_first(h, wmt[0], bm3[0])
    means = [m0]
    rounds = L * STEPS
    for r in range(rounds):
        l = r // 2
        acc2 = _segsum_k(proj.reshape(K * N, D), src2d, dst2d, zeros_nd)
        if r < rounds - 1:
            ln = (r + 1) // 2
            h, colmean, proj = _gru_proj(acc2, h, wih_t[l], whh_t[l],
                                         bih2[l], bhh2[l], wmt[ln], bm3[ln])
        else:
            h, colmean = _gru(acc2, h, wih_t[l], whh_t[l], bih2[l], bhh2[l])
        if r % STEPS == STEPS - 1:
            means.append(colmean)
    agg = jnp.concatenate(means, axis=1)
    res = _head(agg, W1.T, b1.reshape(1, D), W2, b2.reshape(1, 1))
    return (res, agg)
